# baseline (speedup 1.0000x reference)
import functools

import jax
import jax.numpy as jnp
from jax import lax
from jax.experimental import pallas as pl
from jax.experimental.pallas import tpu as pltpu
from jax.experimental.pallas import tpu_sc as plsc

VOCAB = 1000000
N_SEG = 2
MAX_LEN = 200
EMBED = 64
BATCH = 4096
WIDE = 128

N = BATCH * MAX_LEN
NC, NS = 2, 16
NW = NC * NS
ROWS_PER_W = N // NW
CHUNK = 256
NCHUNKS = ROWS_PER_W // CHUNK
TBLOCK = 512
TGRID = (VOCAB + TBLOCK - 1) // TBLOCK


def _widen_body(tokt_ref, wide_ref):
  blk = tokt_ref[...]
  wide_ref[:, :EMBED] = blk.T
  wide_ref[:, EMBED:] = jnp.zeros((TBLOCK, WIDE - EMBED), jnp.float32)


def _body(tok_hbm, comb_hbm, idx_hbm, cidx_hbm, out_hbm,
          comb_s, idx_v, cidx_v, rows0, rows1,
          semc0, semc1, semt0, semt1, semo0, semo1):
  sid = lax.axis_index("s")
  wid = sid * NC + lax.axis_index("c")
  wbase = wid * ROWS_PER_W

  @pl.when(sid == 0)
  def _():
    pltpu.sync_copy(comb_hbm, comb_s)

  pltpu.sync_copy(idx_hbm.at[pl.ds(wbase, ROWS_PER_W)], idx_v)
  pltpu.sync_copy(cidx_hbm.at[pl.ds(wbase, ROWS_PER_W)], cidx_v)
  plsc.subcore_barrier()

  rows = (rows0, rows1)
  semc = (semc0, semc1)
  semt = (semt0, semt1)
  semo = (semo0, semo1)

  def gathers(k, p):
    off = k * CHUNK
    pltpu.async_copy(
        comb_s.at[cidx_v.at[pl.ds(off, CHUNK)]], rows[p], semc[p]).wait()
    pltpu.async_copy(
        tok_hbm.at[idx_v.at[pl.ds(off, CHUNK)]], rows[p], semt[p], add=True)

  def wait_tok(k, p):
    off = k * CHUNK
    pltpu.make_async_copy(
        tok_hbm.at[idx_v.at[pl.ds(off, CHUNK)]], rows[p], semt[p]).wait()

  def store(k, p):
    pltpu.async_copy(
        rows[p], out_hbm.at[pl.ds(wbase + k * CHUNK, CHUNK)], semo[p])

  def wait_store(k, p):
    pltpu.make_async_copy(
        rows[p], out_hbm.at[pl.ds(wbase + k * CHUNK, CHUNK)], semo[p]).wait()

  gathers(0, 0)
  gathers(1, 1)

  def step(j, carry):
    for p in (0, 1):
      k = 2 * j + p
      wait_tok(k, p)
      store(k, p)

      @pl.when(j < NCHUNKS // 2 - 1)
      def _():
        wait_store(k, p)
        gathers(k + 2, p)
    return carry

  lax.fori_loop(0, NCHUNKS // 2, step, 0)
  wait_store(NCHUNKS - 2, 0)
  wait_store(NCHUNKS - 1, 1)


@jax.jit
def _run(tok_table, comb, idx, cidx):
  wide = pl.pallas_call(
      _widen_body,
      grid=(TGRID,),
      in_specs=[pl.BlockSpec((EMBED, TBLOCK), lambda i: (0, i))],
      out_specs=pl.BlockSpec((TBLOCK, WIDE), lambda i: (i, 0)),
      out_shape=jax.ShapeDtypeStruct((VOCAB, WIDE), jnp.float32),
  )(tok_table.T)

  mesh = plsc.VectorSubcoreMesh(core_axis_name="c", subcore_axis_name="s")
  f = pl.kernel(
      _body,
      out_type=jax.ShapeDtypeStruct((N, WIDE), jnp.float32),
      mesh=mesh,
      scratch_types=[
          pltpu.VMEM_SHARED((N_SEG * MAX_LEN, WIDE), jnp.float32),
          pltpu.VMEM((ROWS_PER_W,), jnp.int32),
          pltpu.VMEM((ROWS_PER_W,), jnp.int32),
          pltpu.VMEM((CHUNK, WIDE), jnp.float32),
          pltpu.VMEM((CHUNK, WIDE), jnp.float32),
          pltpu.SemaphoreType.DMA,
          pltpu.SemaphoreType.DMA,
          pltpu.SemaphoreType.DMA,
          pltpu.SemaphoreType.DMA,
          pltpu.SemaphoreType.DMA,
          pltpu.SemaphoreType.DMA,
      ],
      compiler_params=pltpu.CompilerParams(use_tc_tiling_on_sc=False),
  )
  return f(wide, comb, idx, cidx)


def kernel(seq, seg, tok_table, seg_table, pos_table):
  comb = (seg_table[:, None, :] + pos_table[None, :, :]).reshape(
      N_SEG * MAX_LEN, EMBED)
  comb = jnp.pad(comb, ((0, 0), (0, WIDE - EMBED)))
  idx = seq.T.reshape(N)
  cidx = (seg.T * MAX_LEN
          + jnp.arange(MAX_LEN, dtype=jnp.int32)[:, None]).reshape(N)
  out = _run(tok_table, comb, idx, cidx)
  return (out[:, :EMBED].reshape(MAX_LEN, BATCH, EMBED).transpose(1, 0, 2))


# speedup vs baseline: 1.8205x; 1.8205x over previous
import functools

import jax
import jax.numpy as jnp
from jax import lax
from jax.experimental import pallas as pl
from jax.experimental.pallas import tpu as pltpu
from jax.experimental.pallas import tpu_sc as plsc

VOCAB = 1000000
N_SEG = 2
MAX_LEN = 200
EMBED = 64
BATCH = 4096
WIDE = 128

N = BATCH * MAX_LEN
NC, NS = 2, 16
NW = NC * NS
ROWS_PER_W = N // NW
CHUNK = 512
NCHUNKS = ROWS_PER_W // CHUNK


def _body(tok_hbm, comb_hbm, idx_hbm, cidx_hbm, out_hbm,
          comb_s, idx_v, cidx_v, rows0, rows1,
          semc0, semc1, semt0, semt1, semo0, semo1):
  sid = lax.axis_index("s")
  wid = sid * NC + lax.axis_index("c")
  wbase = wid * ROWS_PER_W

  @pl.when(sid == 0)
  def _():
    pltpu.sync_copy(comb_hbm, comb_s)

  pltpu.sync_copy(idx_hbm.at[pl.ds(wbase, ROWS_PER_W)], idx_v)
  pltpu.sync_copy(cidx_hbm.at[pl.ds(wbase, ROWS_PER_W)], cidx_v)
  plsc.subcore_barrier()

  rows = (rows0, rows1)
  semc = (semc0, semc1)
  semt = (semt0, semt1)
  semo = (semo0, semo1)

  def gathers(k, p):
    off = k * CHUNK
    pltpu.async_copy(
        comb_s.at[cidx_v.at[pl.ds(off, CHUNK)]], rows[p], semc[p]).wait()
    pltpu.async_copy(
        tok_hbm.at[idx_v.at[pl.ds(off, CHUNK)]], rows[p], semt[p], add=True)

  def wait_tok(k, p):
    off = k * CHUNK
    pltpu.make_async_copy(
        tok_hbm.at[idx_v.at[pl.ds(off, CHUNK)]], rows[p], semt[p]).wait()

  def store(k, p):
    pltpu.async_copy(
        rows[p],
        out_hbm.at[pl.ds(wbase + k * CHUNK, CHUNK), pl.ds(0, EMBED)],
        semo[p])

  def wait_store(k, p):
    pltpu.make_async_copy(
        rows[p],
        out_hbm.at[pl.ds(wbase + k * CHUNK, CHUNK), pl.ds(0, EMBED)],
        semo[p]).wait()

  gathers(0, 0)
  gathers(1, 1)

  def step(j, carry):
    for p in (0, 1):
      k = 2 * j + p
      wait_tok(k, p)
      store(k, p)

      @pl.when(j < NCHUNKS // 2 - 1)
      def _():
        wait_store(k, p)
        gathers(k + 2, p)
    return carry

  lax.fori_loop(0, NCHUNKS // 2, step, 0)
  wait_store(NCHUNKS - 2, 0)
  wait_store(NCHUNKS - 1, 1)


@jax.jit
def _run(tok_table, comb, idx, cidx):
  mesh = plsc.VectorSubcoreMesh(core_axis_name="c", subcore_axis_name="s")
  f = pl.kernel(
      _body,
      out_type=jax.ShapeDtypeStruct((N, WIDE), jnp.float32),
      mesh=mesh,
      scratch_types=[
          pltpu.VMEM_SHARED((N_SEG * MAX_LEN, EMBED), jnp.float32),
          pltpu.VMEM((ROWS_PER_W,), jnp.int32),
          pltpu.VMEM((ROWS_PER_W,), jnp.int32),
          pltpu.VMEM((CHUNK, EMBED), jnp.float32),
          pltpu.VMEM((CHUNK, EMBED), jnp.float32),
          pltpu.SemaphoreType.DMA,
          pltpu.SemaphoreType.DMA,
          pltpu.SemaphoreType.DMA,
          pltpu.SemaphoreType.DMA,
          pltpu.SemaphoreType.DMA,
          pltpu.SemaphoreType.DMA,
      ],
      compiler_params=pltpu.CompilerParams(use_tc_tiling_on_sc=False),
  )
  return f(tok_table, comb, idx, cidx)


def kernel(seq, seg, tok_table, seg_table, pos_table):
  comb = (seg_table[:, None, :] + pos_table[None, :, :]).reshape(
      N_SEG * MAX_LEN, EMBED)
  idx = seq.T.reshape(N)
  cidx = (seg.T * MAX_LEN
          + jnp.arange(MAX_LEN, dtype=jnp.int32)[:, None]).reshape(N)
  out = _run(tok_table, comb, idx, cidx)
  return (out[:, :EMBED].reshape(MAX_LEN, BATCH, EMBED).transpose(1, 0, 2))
